# baseline (speedup 1.0000x reference)
import functools

import jax
import jax.numpy as jnp
from jax import lax
from jax.experimental import pallas as pl
from jax.experimental.pallas import tpu as pltpu
from jax.experimental.pallas import tpu_sc as plsc

N = 10000
E = 320000
D_IN = 128
D_HID = 16

NC = 2
NS = 16
NW = NC * NS
CH = 80
W = 10
K = 125
CH2 = 125
K2 = 80
R = 10240
RPT = R // NS
GPT = N - 15 * RPT
ZCH = 128

_mesh = plsc.VectorSubcoreMesh(core_axis_name="c", subcore_axis_name="s")
_sc_params = pltpu.CompilerParams(use_tc_tiling_on_sc=False)
_sc_params_nl = pltpu.CompilerParams(use_tc_tiling_on_sc=False,
                                     needs_layout_passes=False)


def _wid():
    return lax.axis_index("s") * NC + lax.axis_index("c")


def _rsqrt_newton(x):
    i = lax.bitcast_convert_type(x, jnp.int32)
    i = jnp.int32(0x5F3759DF) - lax.shift_right_arithmetic(i, 1)
    y = lax.bitcast_convert_type(i, jnp.float32)
    for _ in range(3):
        y = y * (1.5 - 0.5 * x * y * y)
    return y


@functools.partial(
    pl.kernel,
    out_type=jax.ShapeDtypeStruct((NC * R,), jnp.float32),
    mesh=_mesh,
    compiler_params=_sc_params,
    scratch_types=[
        pltpu.VMEM((K2, CH2), jnp.int32),
        pltpu.VMEM((ZCH,), jnp.float32),
        pltpu.VMEM((ZCH,), jnp.float32),
        pltpu.VMEM_SHARED((R,), jnp.float32),
        pltpu.SemaphoreType.DMA,
    ],
)
def _deg_kernel(edge_hbm, out_hbm, dst_v, ones_v, zero_v, acc_sh, sem):
    c = lax.axis_index("c")
    s = lax.axis_index("s")
    w = _wid()
    pltpu.sync_copy(edge_hbm.at[1, w], dst_v)
    for i in range(ZCH // 16):
        ones_v[pl.ds(i * 16, 16)] = jnp.ones((16,), jnp.float32)
    for i in range(ZCH // 16):
        zero_v[pl.ds(i * 16, 16)] = jnp.zeros((16,), jnp.float32)
    for j in range(RPT // ZCH):
        pltpu.sync_copy(zero_v, acc_sh.at[pl.ds(s * RPT + j * ZCH, ZCH)])
    plsc.subcore_barrier()
    descs = [
        pltpu.async_copy(ones_v.at[pl.ds(0, CH2)], acc_sh.at[dst_v.at[g]],
                         sem, add=True)
        for g in range(K2)
    ]
    for dsc in descs:
        dsc.wait()
    plsc.subcore_barrier()
    pltpu.sync_copy(acc_sh.at[pl.ds(s * RPT, RPT)],
                    out_hbm.at[pl.ds(c * R + s * RPT, RPT)])


@functools.partial(
    pl.kernel,
    out_type=(
        jax.ShapeDtypeStruct((NC * R, D_HID), jnp.float32),
        jax.ShapeDtypeStruct((R,), jnp.float32),
    ),
    mesh=_mesh,
    compiler_params=_sc_params_nl,
    scratch_types=[
        pltpu.VMEM((K2, CH2), jnp.int32),
        pltpu.VMEM((K2, CH2), jnp.int32),
        pltpu.VMEM((2 * W, CH2, D_HID), jnp.float32),
        pltpu.VMEM((RPT, D_HID), jnp.float32),
        pltpu.VMEM((RPT,), jnp.float32),
        pltpu.VMEM((RPT,), jnp.float32),
        pltpu.VMEM_SHARED((R, D_HID), jnp.float32),
        pltpu.VMEM_SHARED((R, D_HID), jnp.float32),
        pltpu.SemaphoreType.DMA,
        pltpu.SemaphoreType.DMA,
        pltpu.SemaphoreType.DMA,
        pltpu.SemaphoreType.DMA,
    ],
)
def _agg16_kernel(g_hbm, degp_hbm, edge_hbm, out_hbm, d_hbm,
                  src_v, dst_v, rows_v, grow_v, dtile_v, deg1_v,
                  gs_sh, acc_sh, gsem0, gsem1, ssem0, ssem1):
    c = lax.axis_index("c")
    s = lax.axis_index("s")
    w = _wid()
    gsem = (gsem0, gsem1)
    ssem = (ssem0, ssem1)
    pltpu.sync_copy(edge_hbm.at[0, w], src_v)
    pltpu.sync_copy(edge_hbm.at[1, w], dst_v)

    pltpu.sync_copy(degp_hbm.at[pl.ds(s * RPT, RPT)], dtile_v)
    pltpu.sync_copy(degp_hbm.at[pl.ds(R + s * RPT, RPT)], deg1_v)

    def dbody(i, carry):
        x = dtile_v[pl.ds(i * 16, 16)] + deg1_v[pl.ds(i * 16, 16)] + 1.0
        dtile_v[pl.ds(i * 16, 16)] = _rsqrt_newton(x)
        return carry

    lax.fori_loop(0, RPT // 16, dbody, 0)

    @pl.when(c == 0)
    def _():
        pltpu.sync_copy(dtile_v, d_hbm.at[pl.ds(s * RPT, RPT)])

    @pl.when(s < NS - 1)
    def _():
        pltpu.sync_copy(g_hbm.at[pl.ds(s * RPT, RPT)], grow_v)

    @pl.when(s == NS - 1)
    def _():
        pltpu.sync_copy(g_hbm.at[pl.ds(s * RPT, GPT)],
                        grow_v.at[pl.ds(0, GPT)])

    ridx0 = lax.iota(jnp.int32, 16)

    def sbody(m, carry):
        ridx = ridx0 + m * 16
        dvec = dtile_v[pl.ds(m * 16, 16)]
        for j in range(D_HID):
            cj = jnp.full((16,), j, jnp.int32)
            col = plsc.load_gather(grow_v, [ridx, cj])
            plsc.store_scatter(grow_v, [ridx, cj], col * dvec)
        return carry

    lax.fori_loop(0, RPT // 16, sbody, 0)
    pltpu.sync_copy(grow_v, gs_sh.at[pl.ds(s * RPT, RPT)])

    @pl.when(c == 0)
    def _():
        pltpu.sync_copy(grow_v, acc_sh.at[pl.ds(s * RPT, RPT)])

    @pl.when(c != 0)
    def _():
        def zbody(i, carry):
            grow_v[i, :] = jnp.zeros((16,), jnp.float32)
            return carry

        lax.fori_loop(0, RPT, zbody, 0)
        pltpu.sync_copy(grow_v, acc_sh.at[pl.ds(s * RPT, RPT)])

    plsc.subcore_barrier()

    T = K2 // W

    def slot(g):
        return ((g // W) % 2) * W + g % W

    def start_g(g):
        return pltpu.async_copy(gs_sh.at[src_v.at[g]], rows_v.at[slot(g)],
                                gsem[(g // W) % 2])

    gd = {}
    sd = {}
    for g in range(W):
        gd[g] = start_g(g)
    for t in range(T):
        for b in range(W):
            gd[t * W + b].wait()
        for b in range(W):
            g = t * W + b
            sd[g] = pltpu.async_copy(rows_v.at[slot(g)],
                                     acc_sh.at[dst_v.at[g]],
                                     ssem[t % 2], add=True)
        if t > 0:
            for b in range(W):
                sd[(t - 1) * W + b].wait()
        if t + 1 < T:
            for b in range(W):
                gd[(t + 1) * W + b] = start_g((t + 1) * W + b)
    for b in range(W):
        sd[(T - 1) * W + b].wait()
    plsc.subcore_barrier()
    pltpu.sync_copy(acc_sh.at[pl.ds(s * RPT, RPT)],
                    out_hbm.at[pl.ds(c * R + s * RPT, RPT)])


@functools.partial(
    pl.kernel,
    out_type=jax.ShapeDtypeStruct((NC * R,), jnp.float32),
    mesh=_mesh,
    compiler_params=_sc_params_nl,
    scratch_types=[
        pltpu.VMEM((K, CH), jnp.int32),
        pltpu.VMEM((K, CH), jnp.int32),
        pltpu.VMEM((RPT, D_HID), jnp.float32),
        pltpu.VMEM((RPT, D_HID), jnp.float32),
        pltpu.VMEM((RPT,), jnp.float32),
        pltpu.VMEM((RPT,), jnp.float32),
        pltpu.VMEM((D_HID, 16), jnp.float32),
        pltpu.VMEM((D_HID, 16), jnp.float32),
        pltpu.VMEM((N,), jnp.float32),
        pltpu.VMEM((K * CH,), jnp.float32),
        pltpu.VMEM_SHARED((R,), jnp.float32),
        pltpu.VMEM_SHARED((R,), jnp.float32),
        pltpu.SemaphoreType.DMA,
    ],
)
def _agg1_kernel(accp_hbm, d_hbm, b1_hbm, w2_hbm, edge_hbm, out_hbm,
                 src_v, dst_v, acc0_v, acc1_v, d_v, qs_tile_v, b1_v, w2_v,
                 qs_full_v, rows_v, qs_sh, acc_sh, sem):
    c = lax.axis_index("c")
    s = lax.axis_index("s")
    w = _wid()
    pltpu.sync_copy(edge_hbm.at[0, w], src_v)
    pltpu.sync_copy(edge_hbm.at[1, w], dst_v)
    pltpu.sync_copy(accp_hbm.at[pl.ds(s * RPT, RPT)], acc0_v)
    pltpu.sync_copy(accp_hbm.at[pl.ds(R + s * RPT, RPT)], acc1_v)
    pltpu.sync_copy(d_hbm.at[pl.ds(s * RPT, RPT)], d_v)
    pltpu.sync_copy(b1_hbm, b1_v)
    pltpu.sync_copy(w2_hbm, w2_v)

    ridx0 = lax.iota(jnp.int32, 16)

    def mbody(m, carry):
        ridx = ridx0 + m * 16
        dv = d_v[pl.ds(m * 16, 16)]
        q = jnp.zeros((16,), jnp.float32)
        for j in range(D_HID):
            cj = jnp.full((16,), j, jnp.int32)
            colsum = (plsc.load_gather(acc0_v, [ridx, cj])
                      + plsc.load_gather(acc1_v, [ridx, cj]))
            hj = jnp.maximum(colsum * dv + b1_v[j, :], 0.0)
            q = q + hj * w2_v[j, :]
        qs_tile_v[pl.ds(m * 16, 16)] = q * dv
        return carry

    lax.fori_loop(0, RPT // 16, mbody, 0)
    pltpu.sync_copy(qs_tile_v, qs_sh.at[pl.ds(s * RPT, RPT)])

    @pl.when(c == 0)
    def _():
        pltpu.sync_copy(qs_tile_v, acc_sh.at[pl.ds(s * RPT, RPT)])

    @pl.when(c != 0)
    def _():
        def zbody(i, carry):
            qs_tile_v[pl.ds(i * 16, 16)] = jnp.zeros((16,), jnp.float32)
            return carry

        lax.fori_loop(0, RPT // 16, zbody, 0)
        pltpu.sync_copy(qs_tile_v, acc_sh.at[pl.ds(s * RPT, RPT)])

    plsc.subcore_barrier()
    pltpu.sync_copy(qs_sh.at[pl.ds(0, N)], qs_full_v)

    descs = []
    for g in range(K):
        for i in range(CH // 16):
            idx = src_v[g, pl.ds(i * 16, 16)]
            rows_v[pl.ds(g * CH + i * 16, 16)] = plsc.load_gather(
                qs_full_v, [idx])
        descs.append(
            pltpu.async_copy(rows_v.at[pl.ds(g * CH, CH)],
                             acc_sh.at[dst_v.at[g]], sem, add=True))
    for dsc in descs:
        dsc.wait()
    plsc.subcore_barrier()
    pltpu.sync_copy(acc_sh.at[pl.ds(s * RPT, RPT)],
                    out_hbm.at[pl.ds(c * R + s * RPT, RPT)])


MM_BLK = 1000


def _tc_matmul_body(x_ref, w1_ref, g_ref):
    g_ref[...] = jnp.dot(x_ref[...], w1_ref[...],
                         preferred_element_type=jnp.float32)


def _tc_out_body(acc2p_ref, d_ref, b2_ref, out_ref):
    z = (acc2p_ref[0:N] + acc2p_ref[R:R + N]) * d_ref[0:N] + b2_ref[0]
    out_ref[...] = 1.0 / (1.0 + jnp.exp(-z))


def kernel(x, edge_index, W1, b1, W2, b2):
    edge32 = edge_index.astype(jnp.int32)
    edge_r = edge32.reshape(2, NW, K, CH)
    edge_r2 = edge32.reshape(2, NW, K2, CH2)

    g = pl.pallas_call(
        _tc_matmul_body,
        grid=(N // MM_BLK,),
        in_specs=[
            pl.BlockSpec((MM_BLK, D_IN), lambda i: (i, 0)),
            pl.BlockSpec((D_IN, D_HID), lambda i: (0, 0)),
        ],
        out_specs=pl.BlockSpec((MM_BLK, D_HID), lambda i: (i, 0)),
        out_shape=jax.ShapeDtypeStruct((N, D_HID), jnp.float32),
    )(x, W1)

    degp = _deg_kernel(edge_r2)

    accp, d = _agg16_kernel(g, degp, edge_r2)

    b1r = jnp.tile(b1[:, None], (1, 16))
    w2r = jnp.tile(W2, (1, 16))
    acc2p = _agg1_kernel(accp, d, b1r, w2r, edge_r)

    out = pl.pallas_call(
        _tc_out_body,
        out_shape=jax.ShapeDtypeStruct((N,), jnp.float32),
    )(acc2p, d, b2)
    return out


# speedup vs baseline: 1.0366x; 1.0366x over previous
import functools

import jax
import jax.numpy as jnp
from jax import lax
from jax.experimental import pallas as pl
from jax.experimental.pallas import tpu as pltpu
from jax.experimental.pallas import tpu_sc as plsc

N = 10000
E = 320000
D_IN = 128
D_HID = 16

NC = 2
NS = 16
NW = NC * NS
CH = 80
W = 25
K = 125
R = 10240
RPT = R // NS
GPT = N - 15 * RPT
ZCH = 128

_mesh = plsc.VectorSubcoreMesh(core_axis_name="c", subcore_axis_name="s")
_sc_params = pltpu.CompilerParams(use_tc_tiling_on_sc=False)
_sc_params_nl = pltpu.CompilerParams(use_tc_tiling_on_sc=False,
                                     needs_layout_passes=False)


def _wid():
    return lax.axis_index("s") * NC + lax.axis_index("c")


def _rsqrt_newton(x):
    i = lax.bitcast_convert_type(x, jnp.int32)
    i = jnp.int32(0x5F3759DF) - lax.shift_right_arithmetic(i, 1)
    y = lax.bitcast_convert_type(i, jnp.float32)
    for _ in range(3):
        y = y * (1.5 - 0.5 * x * y * y)
    return y


@functools.partial(
    pl.kernel,
    out_type=jax.ShapeDtypeStruct((NC * R,), jnp.float32),
    mesh=_mesh,
    compiler_params=_sc_params,
    scratch_types=[
        pltpu.VMEM((K, CH), jnp.int32),
        pltpu.VMEM((CH,), jnp.float32),
        pltpu.VMEM((ZCH,), jnp.float32),
        pltpu.VMEM_SHARED((R,), jnp.float32),
        pltpu.SemaphoreType.DMA,
    ],
)
def _deg_kernel(edge_hbm, out_hbm, dst_v, ones_v, zero_v, acc_sh, sem):
    c = lax.axis_index("c")
    s = lax.axis_index("s")
    w = _wid()
    pltpu.sync_copy(edge_hbm.at[1, w], dst_v)
    for i in range(CH // 16):
        ones_v[pl.ds(i * 16, 16)] = jnp.ones((16,), jnp.float32)
    for i in range(ZCH // 16):
        zero_v[pl.ds(i * 16, 16)] = jnp.zeros((16,), jnp.float32)
    for j in range(RPT // ZCH):
        pltpu.sync_copy(zero_v, acc_sh.at[pl.ds(s * RPT + j * ZCH, ZCH)])
    plsc.subcore_barrier()
    descs = [
        pltpu.async_copy(ones_v, acc_sh.at[dst_v.at[g]], sem, add=True)
        for g in range(K)
    ]
    for dsc in descs:
        dsc.wait()
    plsc.subcore_barrier()
    pltpu.sync_copy(acc_sh.at[pl.ds(s * RPT, RPT)],
                    out_hbm.at[pl.ds(c * R + s * RPT, RPT)])


@functools.partial(
    pl.kernel,
    out_type=(
        jax.ShapeDtypeStruct((NC * R, D_HID), jnp.float32),
        jax.ShapeDtypeStruct((R,), jnp.float32),
    ),
    mesh=_mesh,
    compiler_params=_sc_params_nl,
    scratch_types=[
        pltpu.VMEM((K, CH), jnp.int32),
        pltpu.VMEM((K, CH), jnp.int32),
        pltpu.VMEM((2 * W, CH, D_HID), jnp.float32),
        pltpu.VMEM((RPT, D_HID), jnp.float32),
        pltpu.VMEM((RPT,), jnp.float32),
        pltpu.VMEM((RPT,), jnp.float32),
        pltpu.VMEM_SHARED((R, D_HID), jnp.float32),
        pltpu.VMEM_SHARED((R, D_HID), jnp.float32),
        pltpu.SemaphoreType.DMA,
        pltpu.SemaphoreType.DMA,
        pltpu.SemaphoreType.DMA,
        pltpu.SemaphoreType.DMA,
    ],
)
def _agg16_kernel(g_hbm, degp_hbm, edge_hbm, out_hbm, d_hbm,
                  src_v, dst_v, rows_v, grow_v, dtile_v, deg1_v,
                  gs_sh, acc_sh, gsem0, gsem1, ssem0, ssem1):
    c = lax.axis_index("c")
    s = lax.axis_index("s")
    w = _wid()
    gsem = (gsem0, gsem1)
    ssem = (ssem0, ssem1)
    pltpu.sync_copy(edge_hbm.at[0, w], src_v)
    pltpu.sync_copy(edge_hbm.at[1, w], dst_v)

    pltpu.sync_copy(degp_hbm.at[pl.ds(s * RPT, RPT)], dtile_v)
    pltpu.sync_copy(degp_hbm.at[pl.ds(R + s * RPT, RPT)], deg1_v)

    def dbody(i, carry):
        x = dtile_v[pl.ds(i * 16, 16)] + deg1_v[pl.ds(i * 16, 16)] + 1.0
        dtile_v[pl.ds(i * 16, 16)] = _rsqrt_newton(x)
        return carry

    lax.fori_loop(0, RPT // 16, dbody, 0)

    @pl.when(c == 0)
    def _():
        pltpu.sync_copy(dtile_v, d_hbm.at[pl.ds(s * RPT, RPT)])

    @pl.when(s < NS - 1)
    def _():
        pltpu.sync_copy(g_hbm.at[pl.ds(s * RPT, RPT)], grow_v)

    @pl.when(s == NS - 1)
    def _():
        pltpu.sync_copy(g_hbm.at[pl.ds(s * RPT, GPT)],
                        grow_v.at[pl.ds(0, GPT)])

    ridx0 = lax.iota(jnp.int32, 16)

    def sbody(m, carry):
        ridx = ridx0 + m * 16
        dvec = dtile_v[pl.ds(m * 16, 16)]
        for j in range(D_HID):
            cj = jnp.full((16,), j, jnp.int32)
            col = plsc.load_gather(grow_v, [ridx, cj])
            plsc.store_scatter(grow_v, [ridx, cj], col * dvec)
        return carry

    lax.fori_loop(0, RPT // 16, sbody, 0)
    pltpu.sync_copy(grow_v, gs_sh.at[pl.ds(s * RPT, RPT)])

    @pl.when(c == 0)
    def _():
        pltpu.sync_copy(grow_v, acc_sh.at[pl.ds(s * RPT, RPT)])

    @pl.when(c != 0)
    def _():
        def zbody(i, carry):
            grow_v[i, :] = jnp.zeros((16,), jnp.float32)
            return carry

        lax.fori_loop(0, RPT, zbody, 0)
        pltpu.sync_copy(grow_v, acc_sh.at[pl.ds(s * RPT, RPT)])

    plsc.subcore_barrier()

    T = K // W

    def slot(g):
        return ((g // W) % 2) * W + g % W

    def start_g(g):
        return pltpu.async_copy(gs_sh.at[src_v.at[g]], rows_v.at[slot(g)],
                                gsem[(g // W) % 2])

    gd = {}
    sd = {}
    for g in range(W):
        gd[g] = start_g(g)
    for t in range(T):
        for b in range(W):
            gd[t * W + b].wait()
        for b in range(W):
            g = t * W + b
            sd[g] = pltpu.async_copy(rows_v.at[slot(g)],
                                     acc_sh.at[dst_v.at[g]],
                                     ssem[t % 2], add=True)
        if t > 0:
            for b in range(W):
                sd[(t - 1) * W + b].wait()
        if t + 1 < T:
            for b in range(W):
                gd[(t + 1) * W + b] = start_g((t + 1) * W + b)
    for b in range(W):
        sd[(T - 1) * W + b].wait()
    plsc.subcore_barrier()
    pltpu.sync_copy(acc_sh.at[pl.ds(s * RPT, RPT)],
                    out_hbm.at[pl.ds(c * R + s * RPT, RPT)])


@functools.partial(
    pl.kernel,
    out_type=jax.ShapeDtypeStruct((NC * R,), jnp.float32),
    mesh=_mesh,
    compiler_params=_sc_params_nl,
    scratch_types=[
        pltpu.VMEM((K, CH), jnp.int32),
        pltpu.VMEM((K, CH), jnp.int32),
        pltpu.VMEM((RPT, D_HID), jnp.float32),
        pltpu.VMEM((RPT, D_HID), jnp.float32),
        pltpu.VMEM((RPT,), jnp.float32),
        pltpu.VMEM((RPT,), jnp.float32),
        pltpu.VMEM((D_HID, 16), jnp.float32),
        pltpu.VMEM((D_HID, 16), jnp.float32),
        pltpu.VMEM((N,), jnp.float32),
        pltpu.VMEM((K * CH,), jnp.float32),
        pltpu.VMEM_SHARED((R,), jnp.float32),
        pltpu.VMEM_SHARED((R,), jnp.float32),
        pltpu.SemaphoreType.DMA,
    ],
)
def _agg1_kernel(accp_hbm, d_hbm, b1_hbm, w2_hbm, edge_hbm, out_hbm,
                 src_v, dst_v, acc0_v, acc1_v, d_v, qs_tile_v, b1_v, w2_v,
                 qs_full_v, rows_v, qs_sh, acc_sh, sem):
    c = lax.axis_index("c")
    s = lax.axis_index("s")
    w = _wid()
    pltpu.sync_copy(edge_hbm.at[0, w], src_v)
    pltpu.sync_copy(edge_hbm.at[1, w], dst_v)
    pltpu.sync_copy(accp_hbm.at[pl.ds(s * RPT, RPT)], acc0_v)
    pltpu.sync_copy(accp_hbm.at[pl.ds(R + s * RPT, RPT)], acc1_v)
    pltpu.sync_copy(d_hbm.at[pl.ds(s * RPT, RPT)], d_v)
    pltpu.sync_copy(b1_hbm, b1_v)
    pltpu.sync_copy(w2_hbm, w2_v)

    ridx0 = lax.iota(jnp.int32, 16)

    def mbody(m, carry):
        ridx = ridx0 + m * 16
        dv = d_v[pl.ds(m * 16, 16)]
        q = jnp.zeros((16,), jnp.float32)
        for j in range(D_HID):
            cj = jnp.full((16,), j, jnp.int32)
            colsum = (plsc.load_gather(acc0_v, [ridx, cj])
                      + plsc.load_gather(acc1_v, [ridx, cj]))
            hj = jnp.maximum(colsum * dv + b1_v[j, :], 0.0)
            q = q + hj * w2_v[j, :]
        qs_tile_v[pl.ds(m * 16, 16)] = q * dv
        return carry

    lax.fori_loop(0, RPT // 16, mbody, 0)
    pltpu.sync_copy(qs_tile_v, qs_sh.at[pl.ds(s * RPT, RPT)])

    @pl.when(c == 0)
    def _():
        pltpu.sync_copy(qs_tile_v, acc_sh.at[pl.ds(s * RPT, RPT)])

    @pl.when(c != 0)
    def _():
        def zbody(i, carry):
            qs_tile_v[pl.ds(i * 16, 16)] = jnp.zeros((16,), jnp.float32)
            return carry

        lax.fori_loop(0, RPT // 16, zbody, 0)
        pltpu.sync_copy(qs_tile_v, acc_sh.at[pl.ds(s * RPT, RPT)])

    plsc.subcore_barrier()
    pltpu.sync_copy(qs_sh.at[pl.ds(0, N)], qs_full_v)

    descs = []
    for g in range(K):
        for i in range(CH // 16):
            idx = src_v[g, pl.ds(i * 16, 16)]
            rows_v[pl.ds(g * CH + i * 16, 16)] = plsc.load_gather(
                qs_full_v, [idx])
        descs.append(
            pltpu.async_copy(rows_v.at[pl.ds(g * CH, CH)],
                             acc_sh.at[dst_v.at[g]], sem, add=True))
    for dsc in descs:
        dsc.wait()
    plsc.subcore_barrier()
    pltpu.sync_copy(acc_sh.at[pl.ds(s * RPT, RPT)],
                    out_hbm.at[pl.ds(c * R + s * RPT, RPT)])


MM_BLK = 1000


def _tc_matmul_body(x_ref, w1_ref, g_ref):
    g_ref[...] = jnp.dot(x_ref[...], w1_ref[...],
                         preferred_element_type=jnp.float32)


def _tc_out_body(acc2p_ref, d_ref, b2_ref, out_ref):
    z = (acc2p_ref[0:N] + acc2p_ref[R:R + N]) * d_ref[0:N] + b2_ref[0]
    out_ref[...] = 1.0 / (1.0 + jnp.exp(-z))


def kernel(x, edge_index, W1, b1, W2, b2):
    edge_r = edge_index.astype(jnp.int32).reshape(2, NW, K, CH)

    g = pl.pallas_call(
        _tc_matmul_body,
        grid=(N // MM_BLK,),
        in_specs=[
            pl.BlockSpec((MM_BLK, D_IN), lambda i: (i, 0)),
            pl.BlockSpec((D_IN, D_HID), lambda i: (0, 0)),
        ],
        out_specs=pl.BlockSpec((MM_BLK, D_HID), lambda i: (i, 0)),
        out_shape=jax.ShapeDtypeStruct((N, D_HID), jnp.float32),
    )(x, W1)

    degp = _deg_kernel(edge_r)

    accp, d = _agg16_kernel(g, degp, edge_r)

    b1r = jnp.tile(b1[:, None], (1, 16))
    w2r = jnp.tile(W2, (1, 16))
    acc2p = _agg1_kernel(accp, d, b1r, w2r, edge_r)

    out = pl.pallas_call(
        _tc_out_body,
        out_shape=jax.ShapeDtypeStruct((N,), jnp.float32),
    )(acc2p, d, b2)
    return out
